# baseline (speedup 1.0000x reference)
import functools

import jax
import jax.numpy as jnp
from jax import lax
from jax.experimental import pallas as pl
from jax.experimental.pallas import tpu as pltpu
from jax.experimental.pallas import tpu_sc as plsc

N = 10000
E = 160000
B = 128

NC = 2
NS = 16
NW = NC * NS
PER_W = E // NW
ROWS_PER_SUB = N // NS

TILE_E = 2000
TILE_N = 2000


def _lsilu(x, alpha):
    u = jnp.clip(0.5 * x, -3.0, 3.0)
    u2 = u * u
    t = u * (27.0 + u2) / (27.0 + 9.0 * u2)
    return x * (0.5 + 0.5 * t + alpha)


def _bn_fold(bn):
    k = bn["g"] / jnp.sqrt(bn["rv"] + 1e-5)
    return k, bn["be"] - bn["rm"] * k


def _fold_conv(p, din):
    k1, s1 = _bn_fold(p["bn1"])
    w1 = (p["l1"]["w"] * k1[None, :]).astype(jnp.bfloat16)
    b1 = p["l1"]["b"] * k1 + s1
    k2, s2 = _bn_fold(p["bn2"])
    w2 = (p["l2"]["w"] * k2[None, :]).astype(jnp.bfloat16)
    b2 = p["l2"]["b"] * k2 + s2
    return w1[:din], w1[din:], b1[None, :], w2, b2[None, :]


def _embed_call(x, batch_col, m0, m1, m2):
    grid = N // TILE_N

    def body(x_ref, b_ref, m0_ref, m1_ref, m2_ref, o_ref):
        xt = x_ref[...]
        bt = b_ref[...]
        iota = lax.broadcasted_iota(jnp.int32, (1, B), 1).astype(jnp.float32)
        oh = jnp.where(bt == iota, 1.0, 0.0)
        col0 = lax.broadcasted_iota(jnp.int32, (1, 4), 1) == 0
        acc = xt[:, 0:1] * jnp.where(col0, 1.0, 0.0)
        for k, m_ref in enumerate((m0_ref, m1_ref, m2_ref)):
            mk = jnp.dot(oh, m_ref[...], preferred_element_type=jnp.float32)
            acc = acc + xt[:, 1 + k:2 + k] * mk
        o_ref[...] = acc

    return pl.pallas_call(
        body,
        grid=(grid,),
        in_specs=[
            pl.BlockSpec((TILE_N, 4), lambda i: (i, 0)),
            pl.BlockSpec((TILE_N, 1), lambda i: (i, 0)),
            pl.BlockSpec((B, 4), lambda i: (0, 0)),
            pl.BlockSpec((B, 4), lambda i: (0, 0)),
            pl.BlockSpec((B, 4), lambda i: (0, 0)),
        ],
        out_specs=pl.BlockSpec((TILE_N, 4), lambda i: (i, 0)),
        out_shape=jax.ShapeDtypeStruct((N, 4), jnp.float32),
    )(x, batch_col, m0, m1, m2)


def _edge_mlp_call(xd, xs, w1d, w1s, b1, w2, b2):
    din = xd.shape[1]
    h_dim = w1d.shape[1]
    dout = w2.shape[1]
    e_cnt = xd.shape[0]
    grid = e_cnt // TILE_E

    def body(xd_ref, xs_ref, w1d_ref, w1s_ref, b1_ref, w2_ref, b2_ref, o_ref):
        h = jnp.dot(xd_ref[...].astype(jnp.bfloat16), w1d_ref[...],
                    preferred_element_type=jnp.float32)
        h = h + jnp.dot(xs_ref[...].astype(jnp.bfloat16), w1s_ref[...],
                        preferred_element_type=jnp.float32)
        h = _lsilu(h + b1_ref[...], 0.05)
        m = jnp.dot(h.astype(jnp.bfloat16), w2_ref[...],
                    preferred_element_type=jnp.float32) + b2_ref[...]
        o_ref[...] = _lsilu(m, 0.05)

    return pl.pallas_call(
        body,
        grid=(grid,),
        in_specs=[
            pl.BlockSpec((TILE_E, din), lambda i: (i, 0)),
            pl.BlockSpec((TILE_E, din), lambda i: (i, 0)),
            pl.BlockSpec((din, h_dim), lambda i: (0, 0)),
            pl.BlockSpec((din, h_dim), lambda i: (0, 0)),
            pl.BlockSpec((1, h_dim), lambda i: (0, 0)),
            pl.BlockSpec((h_dim, dout), lambda i: (0, 0)),
            pl.BlockSpec((1, dout), lambda i: (0, 0)),
        ],
        out_specs=pl.BlockSpec((TILE_E, dout), lambda i: (i, 0)),
        out_shape=jax.ShapeDtypeStruct((e_cnt, dout), jnp.float32),
    )(xd, xs, w1d, w1s, b1, w2, b2)


def _node_update_call(part, cnt, w1, b1, w2, b2, scale, shift):
    d = part.shape[2]
    h_dim = w1.shape[1]
    grid = N // TILE_N

    def body(p_ref, c_ref, w1_ref, b1_ref, w2_ref, b2_ref, sc_ref, sh_ref,
             o_ref):
        s = p_ref[0] + p_ref[1]
        c = c_ref[0, :, 0:1] + c_ref[1, :, 0:1]
        mean = s * (1.0 / jnp.maximum(c, 1.0))
        nf0 = _lsilu(mean, 0.1)
        h = _lsilu(jnp.dot(nf0.astype(jnp.bfloat16), w1_ref[...],
                           preferred_element_type=jnp.float32) + b1_ref[...], 0.05)
        t = _lsilu(jnp.dot(h.astype(jnp.bfloat16), w2_ref[...],
                           preferred_element_type=jnp.float32) + b2_ref[...], 0.05)
        o_ref[...] = t * sc_ref[...] + sh_ref[...] + nf0

    return pl.pallas_call(
        body,
        grid=(grid,),
        in_specs=[
            pl.BlockSpec((2, TILE_N, d), lambda i: (0, i, 0)),
            pl.BlockSpec((2, TILE_N, 8), lambda i: (0, i, 0)),
            pl.BlockSpec((d, h_dim), lambda i: (0, 0)),
            pl.BlockSpec((1, h_dim), lambda i: (0, 0)),
            pl.BlockSpec((h_dim, d), lambda i: (0, 0)),
            pl.BlockSpec((1, d), lambda i: (0, 0)),
            pl.BlockSpec((1, d), lambda i: (0, 0)),
            pl.BlockSpec((1, d), lambda i: (0, 0)),
        ],
        out_specs=pl.BlockSpec((TILE_N, d), lambda i: (i, 0)),
        out_shape=jax.ShapeDtypeStruct((N, d), jnp.float32),
    )(part, cnt, w1, b1, w2, b2, scale, shift)


def _pool_fc_call(part, cnt, batch_col, fw1, fb1, fw2, fb2):
    d = part.shape[2]
    grid = N // TILE_N

    def body(p_ref, c_ref, b_ref, fw1_ref, fb1_ref, fw2_ref, fb2_ref,
             o_ref, acc):
        i = pl.program_id(0)
        s = p_ref[0] + p_ref[1]
        c = c_ref[0, :, 0:1] + c_ref[1, :, 0:1]
        nf = _lsilu(s * (1.0 / jnp.maximum(c, 1.0)), 0.1)
        bt = b_ref[...]
        iota = lax.broadcasted_iota(jnp.int32, (1, B), 1).astype(jnp.float32)
        oh = jnp.where(bt == iota, 1.0, 0.0)
        g = lax.dot_general(oh, nf, (((0,), (0,)), ((), ())),
                            preferred_element_type=jnp.float32)

        @pl.when(i == 0)
        def _():
            acc[...] = jnp.zeros_like(acc)

        acc[...] += g

        @pl.when(i == grid - 1)
        def _():
            h = jnp.dot(acc[...], fw1_ref[...], preferred_element_type=jnp.float32)
            h = _lsilu(h + fb1_ref[...], 0.1)
            o_ref[...] = (jnp.dot(h, fw2_ref[...], preferred_element_type=jnp.float32)
                          + fb2_ref[...] - 100.0)

    return pl.pallas_call(
        body,
        grid=(grid,),
        in_specs=[
            pl.BlockSpec((2, TILE_N, d), lambda i: (0, i, 0)),
            pl.BlockSpec((2, TILE_N, 8), lambda i: (0, i, 0)),
            pl.BlockSpec((TILE_N, 1), lambda i: (i, 0)),
            pl.BlockSpec((d, d), lambda i: (0, 0)),
            pl.BlockSpec((1, d), lambda i: (0, 0)),
            pl.BlockSpec((d, 1), lambda i: (0, 0)),
            pl.BlockSpec((1, 1), lambda i: (0, 0)),
        ],
        out_specs=pl.BlockSpec((B, 1), lambda i: (0, 0)),
        out_shape=jax.ShapeDtypeStruct((B, 1), jnp.float32),
        scratch_shapes=[pltpu.VMEM((B, d), jnp.float32)],
    )(part, cnt, batch_col, fw1, fb1, fw2, fb2)


def _sc_gather(nf, src, dst, e0, e_cnt):
    d = nf.shape[1]
    dt = nf.dtype
    per_w = e_cnt // NW
    row_bytes = d * jnp.dtype(dt).itemsize
    chunk = 1000 if row_bytes <= 64 else 200
    iters = per_w // chunk
    odd = iters % 2 == 1
    g_lim = (iters - 3) // 2 if odd else (iters - 2) // 2
    tc_tiled = (row_bytes == 512)
    mesh = plsc.VectorSubcoreMesh(core_axis_name="c", subcore_axis_name="s")

    @functools.partial(
        pl.kernel,
        out_type=(jax.ShapeDtypeStruct((e_cnt, d), dt),
                  jax.ShapeDtypeStruct((e_cnt, d), dt)),
        mesh=mesh,
        compiler_params=pltpu.CompilerParams(use_tc_tiling_on_sc=tc_tiled),
        scratch_types=[
            pltpu.VMEM((per_w,), jnp.int32),
            pltpu.VMEM((per_w,), jnp.int32),
            pltpu.VMEM((chunk, d), dt),
            pltpu.VMEM((chunk, d), dt),
            pltpu.VMEM((chunk, d), dt),
            pltpu.VMEM((chunk, d), dt),
            pltpu.SemaphoreType.DMA,
            pltpu.SemaphoreType.DMA,
            pltpu.SemaphoreType.DMA,
            pltpu.SemaphoreType.DMA,
        ],
    )
    def gk(nf_hbm, src_hbm, dst_hbm, xd_out, xs_out,
           idx_d, idx_s, rd0, rd1, rs0, rs1, sd0, sd1, ss0, ss1):
        wid = lax.axis_index("s") * NC + lax.axis_index("c")
        base = wid * per_w
        rd = (rd0, rd1)
        rs = (rs0, rs1)
        sd = (sd0, sd1)
        ss = (ss0, ss1)
        pltpu.sync_copy(dst_hbm.at[pl.ds(e0 + base, per_w)], idx_d)
        pltpu.sync_copy(src_hbm.at[pl.ds(e0 + base, per_w)], idx_s)

        def fire(i, b):
            pltpu.async_copy(nf_hbm.at[idx_d.at[pl.ds(i * chunk, chunk)]],
                             rd[b], sd[b])
            pltpu.async_copy(nf_hbm.at[idx_s.at[pl.ds(i * chunk, chunk)]],
                             rs[b], ss[b])

        def wait(i, b):
            pltpu.make_async_copy(nf_hbm.at[idx_d.at[pl.ds(i * chunk, chunk)]],
                                  rd[b], sd[b]).wait()
            pltpu.make_async_copy(nf_hbm.at[idx_s.at[pl.ds(i * chunk, chunk)]],
                                  rs[b], ss[b]).wait()

        def writeback(i, b):
            off = base + i * chunk
            pltpu.sync_copy(rd[b], xd_out.at[pl.ds(off, chunk)])
            pltpu.sync_copy(rs[b], xs_out.at[pl.ds(off, chunk)])

        fire(0, 0)
        fire(1, 1)

        def body(g, carry):
            i0 = 2 * g
            wait(i0, 0)
            writeback(i0, 0)
            fire(i0 + 2, 0)
            wait(i0 + 1, 1)
            writeback(i0 + 1, 1)
            fire(i0 + 3, 1)
            return carry

        lax.fori_loop(0, g_lim, body, 0)
        i0 = 2 * g_lim
        wait(i0, 0)
        writeback(i0, 0)
        if odd:
            fire(i0 + 2, 0)
        wait(i0 + 1, 1)
        writeback(i0 + 1, 1)
        if odd:
            wait(i0 + 2, 0)
            writeback(i0 + 2, 0)

    return gk(nf, src, dst)


def _sc_scatter(msg, dst3, zeros):
    d = msg.shape[1]
    dt = msg.dtype
    iters, chunk = dst3.shape[1], dst3.shape[2]
    odd = iters % 2 == 1
    g_lim = (iters - 3) // 2 if odd else (iters - 2) // 2
    per_w = iters * chunk
    row_bytes = d * jnp.dtype(dt).itemsize
    tc_tiled = (row_bytes == 512)
    n_acc = 10240 if tc_tiled else N
    rps = n_acc // NS
    mesh = plsc.VectorSubcoreMesh(core_axis_name="c", subcore_axis_name="s")

    @functools.partial(
        pl.kernel,
        out_type=jax.ShapeDtypeStruct((NC, N, d), dt),
        mesh=mesh,
        compiler_params=pltpu.CompilerParams(use_tc_tiling_on_sc=tc_tiled),
        scratch_types=[
            pltpu.VMEM((iters, chunk), jnp.int32),
            pltpu.VMEM((chunk, d), dt),
            pltpu.VMEM((chunk, d), dt),
            pltpu.VMEM_SHARED((n_acc, d), dt),
            pltpu.SemaphoreType.DMA,
            pltpu.SemaphoreType.DMA,
        ],
    )
    def sk(msg_hbm, dst_hbm, z_hbm, out_hbm, idx_v, m0, m1, acc, sm0, sm1):
        cid = lax.axis_index("c")
        sid = lax.axis_index("s")
        wid = sid * NC + cid
        r0 = sid * rps
        pltpu.sync_copy(dst_hbm.at[wid], idx_v)
        pltpu.sync_copy(z_hbm.at[pl.ds(r0, rps)],
                        acc.at[pl.ds(r0, rps)])
        plsc.subcore_barrier()
        base = wid * per_w
        m = (m0, m1)
        sm = (sm0, sm1)

        def fire(i, b):
            pltpu.async_copy(msg_hbm.at[pl.ds(base + i * chunk, chunk)],
                             m[b], sm[b])

        def wait(i, b):
            pltpu.make_async_copy(msg_hbm.at[pl.ds(base + i * chunk, chunk)],
                                  m[b], sm[b]).wait()

        def add(i, b):
            pltpu.sync_copy(m[b], acc.at[idx_v.at[i]], add=True)

        fire(0, 0)
        fire(1, 1)

        def body(g, carry):
            i0 = 2 * g
            wait(i0, 0)
            add(i0, 0)
            fire(i0 + 2, 0)
            wait(i0 + 1, 1)
            add(i0 + 1, 1)
            fire(i0 + 3, 1)
            return carry

        lax.fori_loop(0, g_lim, body, 0)
        i0 = 2 * g_lim
        wait(i0, 0)
        add(i0, 0)
        if odd:
            fire(i0 + 2, 0)
        wait(i0 + 1, 1)
        add(i0 + 1, 1)
        if odd:
            wait(i0 + 2, 0)
            add(i0 + 2, 0)
        plsc.subcore_barrier()
        if n_acc == N:
            pltpu.sync_copy(acc.at[pl.ds(r0, rps)],
                            out_hbm.at[cid, pl.ds(r0, rps)])
        else:
            last = N - (NS - 1) * rps

            @pl.when(sid < NS - 1)
            def _():
                pltpu.sync_copy(acc.at[pl.ds(r0, rps)],
                                out_hbm.at[cid, pl.ds(r0, rps)])

            @pl.when(sid == NS - 1)
            def _():
                pltpu.sync_copy(acc.at[pl.ds((NS - 1) * rps, last)],
                                out_hbm.at[cid, pl.ds((NS - 1) * rps, last)])

    return sk(msg, dst3, zeros)


def _sc_counts(dst, ones, zeros):
    chunk = 1000
    iters = PER_W // chunk
    mesh = plsc.VectorSubcoreMesh(core_axis_name="c", subcore_axis_name="s")

    @functools.partial(
        pl.kernel,
        out_type=jax.ShapeDtypeStruct((NC, N, 8), jnp.float32),
        mesh=mesh,
        compiler_params=pltpu.CompilerParams(use_tc_tiling_on_sc=False),
        scratch_types=[
            pltpu.VMEM((chunk,), jnp.int32),
            pltpu.VMEM((chunk, 8), jnp.float32),
            pltpu.VMEM_SHARED((N, 8), jnp.float32),
        ],
    )
    def ck(dst_hbm, ones_hbm, z_hbm, out_hbm, idx_v, ones_v, acc):
        cid = lax.axis_index("c")
        sid = lax.axis_index("s")
        wid = sid * NC + cid
        r0 = sid * ROWS_PER_SUB
        pltpu.sync_copy(ones_hbm, ones_v)
        pltpu.sync_copy(z_hbm.at[pl.ds(r0, ROWS_PER_SUB)],
                        acc.at[pl.ds(r0, ROWS_PER_SUB)])
        plsc.subcore_barrier()
        base = wid * PER_W

        def body(i, carry):
            off = base + i * chunk
            pltpu.sync_copy(dst_hbm.at[pl.ds(off, chunk)], idx_v)
            pltpu.sync_copy(ones_v, acc.at[idx_v], add=True)
            return carry

        lax.fori_loop(0, iters, body, 0)
        plsc.subcore_barrier()
        pltpu.sync_copy(acc.at[pl.ds(r0, ROWS_PER_SUB)],
                        out_hbm.at[cid, pl.ds(r0, ROWS_PER_SUB)])

    return ck(dst, ones, zeros)


E_A = 96000
E_B = 64000

_SPLIT = ((0, E),)


def _conv_layer(nf, src, dst, p, din):
    w1d, w1s, b1, w2, b2 = _fold_conv(p, din)
    dout = w2.shape[1]
    chunk = {32: 1000, 64: 200, 128: 40}[dout]
    n_acc = 10240 if dout == 128 else N
    parts = []
    for e0, e_cnt in _SPLIT:
        xd, xs = _sc_gather(nf, src, dst, e0, e_cnt)
        msg = _edge_mlp_call(xd, xs, w1d, w1s, b1, w2, b2)
        per_w = e_cnt // NW
        dst3 = lax.dynamic_slice_in_dim(dst, e0, e_cnt).reshape(
            NW, per_w // chunk, chunk)
        parts.append(_sc_scatter(msg, dst3, jnp.zeros((n_acc, dout), msg.dtype)))
    return parts


def kernel(x, matrix, params, batch, edge_index):
    src = edge_index[0]
    dst = edge_index[1]
    batch_col = batch.astype(jnp.float32).reshape(N, 1)

    mg = matrix.astype(jnp.float32).reshape(B, 3, 3)
    zpad = jnp.zeros((B, 1), jnp.float32)
    m0 = jnp.concatenate([zpad, mg[:, 0, :]], axis=1)
    m1 = jnp.concatenate([zpad, mg[:, 1, :]], axis=1)
    m2 = jnp.concatenate([zpad, mg[:, 2, :]], axis=1)

    nf = _embed_call(x, batch_col, m0, m1, m2)

    cnt = _sc_counts(dst, jnp.ones((1000, 8), jnp.float32),
                     jnp.zeros((N, 8), jnp.float32))

    dims = (4, 32, 128, 64, 128)
    for li in range(4):
        p = params[f"conv{li + 1}"]
        (pa,) = _conv_layer(nf, src, dst, p, dims[li])
        ip = params[f"il{li + 1}"]
        k, s = _bn_fold(ip["bn"])
        nf = _node_update_call(
            pa, cnt,
            ip["l1"]["w"].astype(jnp.bfloat16), ip["l1"]["b"][None, :],
            ip["l2"]["w"].astype(jnp.bfloat16), ip["l2"]["b"][None, :],
            k[None, :], s[None, :])

    (pa,) = _conv_layer(nf, src, dst, params["conv5"], dims[4])
    return _pool_fc_call(pa, cnt, batch_col,
                         params["fc1"]["w"], params["fc1"]["b"][None, :],
                         params["fc2"]["w"], params["fc2"]["b"][None, :])


# speedup vs baseline: 1.0777x; 1.0777x over previous
import functools

import jax
import jax.numpy as jnp
from jax import lax
from jax.experimental import pallas as pl
from jax.experimental.pallas import tpu as pltpu
from jax.experimental.pallas import tpu_sc as plsc

N = 10000
E = 160000
B = 128

NC = 2
NS = 16
NW = NC * NS
PER_W = E // NW
ROWS_PER_SUB = N // NS

TILE_E = 2000
TILE_N = 2000


def _lsilu(x, alpha):
    return x * (1.0 / (1.0 + jnp.exp(-x)) + alpha)


def _bn_fold(bn):
    k = bn["g"] / jnp.sqrt(bn["rv"] + 1e-5)
    return k, bn["be"] - bn["rm"] * k


def _fold_conv(p, din):
    k1, s1 = _bn_fold(p["bn1"])
    w1 = (p["l1"]["w"] * k1[None, :]).astype(jnp.bfloat16)
    b1 = p["l1"]["b"] * k1 + s1
    k2, s2 = _bn_fold(p["bn2"])
    w2 = (p["l2"]["w"] * k2[None, :]).astype(jnp.bfloat16)
    b2 = p["l2"]["b"] * k2 + s2
    return w1[:din], w1[din:], b1[None, :], w2, b2[None, :]


def _embed_call(x, batch_col, m0, m1, m2):
    grid = N // TILE_N

    def body(x_ref, b_ref, m0_ref, m1_ref, m2_ref, o_ref):
        xt = x_ref[...]
        bt = b_ref[...]
        iota = lax.broadcasted_iota(jnp.int32, (1, B), 1).astype(jnp.float32)
        oh = jnp.where(bt == iota, 1.0, 0.0)
        col0 = lax.broadcasted_iota(jnp.int32, (1, 4), 1) == 0
        acc = xt[:, 0:1] * jnp.where(col0, 1.0, 0.0)
        for k, m_ref in enumerate((m0_ref, m1_ref, m2_ref)):
            mk = jnp.dot(oh, m_ref[...], preferred_element_type=jnp.float32)
            acc = acc + xt[:, 1 + k:2 + k] * mk
        o_ref[...] = acc

    return pl.pallas_call(
        body,
        grid=(grid,),
        in_specs=[
            pl.BlockSpec((TILE_N, 4), lambda i: (i, 0)),
            pl.BlockSpec((TILE_N, 1), lambda i: (i, 0)),
            pl.BlockSpec((B, 4), lambda i: (0, 0)),
            pl.BlockSpec((B, 4), lambda i: (0, 0)),
            pl.BlockSpec((B, 4), lambda i: (0, 0)),
        ],
        out_specs=pl.BlockSpec((TILE_N, 4), lambda i: (i, 0)),
        out_shape=jax.ShapeDtypeStruct((N, 4), jnp.float32),
    )(x, batch_col, m0, m1, m2)


def _edge_mlp_call(xd, xs, w1d, w1s, b1, w2, b2):
    din = xd.shape[1]
    h_dim = w1d.shape[1]
    dout = w2.shape[1]
    e_cnt = xd.shape[0]
    grid = e_cnt // TILE_E

    def body(xd_ref, xs_ref, w1d_ref, w1s_ref, b1_ref, w2_ref, b2_ref, o_ref):
        h = jnp.dot(xd_ref[...].astype(jnp.bfloat16), w1d_ref[...],
                    preferred_element_type=jnp.float32)
        h = h + jnp.dot(xs_ref[...].astype(jnp.bfloat16), w1s_ref[...],
                        preferred_element_type=jnp.float32)
        h = _lsilu(h + b1_ref[...], 0.05)
        m = jnp.dot(h.astype(jnp.bfloat16), w2_ref[...],
                    preferred_element_type=jnp.float32) + b2_ref[...]
        o_ref[...] = _lsilu(m, 0.05)

    return pl.pallas_call(
        body,
        grid=(grid,),
        in_specs=[
            pl.BlockSpec((TILE_E, din), lambda i: (i, 0)),
            pl.BlockSpec((TILE_E, din), lambda i: (i, 0)),
            pl.BlockSpec((din, h_dim), lambda i: (0, 0)),
            pl.BlockSpec((din, h_dim), lambda i: (0, 0)),
            pl.BlockSpec((1, h_dim), lambda i: (0, 0)),
            pl.BlockSpec((h_dim, dout), lambda i: (0, 0)),
            pl.BlockSpec((1, dout), lambda i: (0, 0)),
        ],
        out_specs=pl.BlockSpec((TILE_E, dout), lambda i: (i, 0)),
        out_shape=jax.ShapeDtypeStruct((e_cnt, dout), jnp.float32),
    )(xd, xs, w1d, w1s, b1, w2, b2)


def _node_update_call(part, cnt, w1, b1, w2, b2, scale, shift):
    d = part.shape[2]
    h_dim = w1.shape[1]
    grid = N // TILE_N

    def body(p_ref, c_ref, w1_ref, b1_ref, w2_ref, b2_ref, sc_ref, sh_ref,
             o_ref):
        s = p_ref[0] + p_ref[1]
        c = c_ref[0, :, 0:1] + c_ref[1, :, 0:1]
        mean = s * (1.0 / jnp.maximum(c, 1.0))
        nf0 = _lsilu(mean, 0.1)
        h = _lsilu(jnp.dot(nf0.astype(jnp.bfloat16), w1_ref[...],
                           preferred_element_type=jnp.float32) + b1_ref[...], 0.05)
        t = _lsilu(jnp.dot(h.astype(jnp.bfloat16), w2_ref[...],
                           preferred_element_type=jnp.float32) + b2_ref[...], 0.05)
        o_ref[...] = t * sc_ref[...] + sh_ref[...] + nf0

    return pl.pallas_call(
        body,
        grid=(grid,),
        in_specs=[
            pl.BlockSpec((2, TILE_N, d), lambda i: (0, i, 0)),
            pl.BlockSpec((2, TILE_N, 8), lambda i: (0, i, 0)),
            pl.BlockSpec((d, h_dim), lambda i: (0, 0)),
            pl.BlockSpec((1, h_dim), lambda i: (0, 0)),
            pl.BlockSpec((h_dim, d), lambda i: (0, 0)),
            pl.BlockSpec((1, d), lambda i: (0, 0)),
            pl.BlockSpec((1, d), lambda i: (0, 0)),
            pl.BlockSpec((1, d), lambda i: (0, 0)),
        ],
        out_specs=pl.BlockSpec((TILE_N, d), lambda i: (i, 0)),
        out_shape=jax.ShapeDtypeStruct((N, d), jnp.float32),
    )(part, cnt, w1, b1, w2, b2, scale, shift)


def _pool_fc_call(part, cnt, batch_col, fw1, fb1, fw2, fb2):
    d = part.shape[2]
    grid = N // TILE_N

    def body(p_ref, c_ref, b_ref, fw1_ref, fb1_ref, fw2_ref, fb2_ref,
             o_ref, acc):
        i = pl.program_id(0)
        s = p_ref[0] + p_ref[1]
        c = c_ref[0, :, 0:1] + c_ref[1, :, 0:1]
        nf = _lsilu(s * (1.0 / jnp.maximum(c, 1.0)), 0.1)
        bt = b_ref[...]
        iota = lax.broadcasted_iota(jnp.int32, (1, B), 1).astype(jnp.float32)
        oh = jnp.where(bt == iota, 1.0, 0.0)
        g = lax.dot_general(oh, nf, (((0,), (0,)), ((), ())),
                            preferred_element_type=jnp.float32)

        @pl.when(i == 0)
        def _():
            acc[...] = jnp.zeros_like(acc)

        acc[...] += g

        @pl.when(i == grid - 1)
        def _():
            h = jnp.dot(acc[...], fw1_ref[...], preferred_element_type=jnp.float32)
            h = _lsilu(h + fb1_ref[...], 0.1)
            o_ref[...] = (jnp.dot(h, fw2_ref[...], preferred_element_type=jnp.float32)
                          + fb2_ref[...] - 100.0)

    return pl.pallas_call(
        body,
        grid=(grid,),
        in_specs=[
            pl.BlockSpec((2, TILE_N, d), lambda i: (0, i, 0)),
            pl.BlockSpec((2, TILE_N, 8), lambda i: (0, i, 0)),
            pl.BlockSpec((TILE_N, 1), lambda i: (i, 0)),
            pl.BlockSpec((d, d), lambda i: (0, 0)),
            pl.BlockSpec((1, d), lambda i: (0, 0)),
            pl.BlockSpec((d, 1), lambda i: (0, 0)),
            pl.BlockSpec((1, 1), lambda i: (0, 0)),
        ],
        out_specs=pl.BlockSpec((B, 1), lambda i: (0, 0)),
        out_shape=jax.ShapeDtypeStruct((B, 1), jnp.float32),
        scratch_shapes=[pltpu.VMEM((B, d), jnp.float32)],
    )(part, cnt, batch_col, fw1, fb1, fw2, fb2)


def _sc_gather(nf, src, dst, e0, e_cnt):
    d = nf.shape[1]
    dt = nf.dtype
    per_w = e_cnt // NW
    row_bytes = d * jnp.dtype(dt).itemsize
    chunk = 1000 if row_bytes <= 64 else 200
    iters = per_w // chunk
    odd = iters % 2 == 1
    g_lim = (iters - 3) // 2 if odd else (iters - 2) // 2
    tc_tiled = (row_bytes == 512)
    mesh = plsc.VectorSubcoreMesh(core_axis_name="c", subcore_axis_name="s")

    @functools.partial(
        pl.kernel,
        out_type=(jax.ShapeDtypeStruct((e_cnt, d), dt),
                  jax.ShapeDtypeStruct((e_cnt, d), dt)),
        mesh=mesh,
        compiler_params=pltpu.CompilerParams(use_tc_tiling_on_sc=tc_tiled),
        scratch_types=[
            pltpu.VMEM((per_w,), jnp.int32),
            pltpu.VMEM((per_w,), jnp.int32),
            pltpu.VMEM((chunk, d), dt),
            pltpu.VMEM((chunk, d), dt),
            pltpu.VMEM((chunk, d), dt),
            pltpu.VMEM((chunk, d), dt),
            pltpu.SemaphoreType.DMA,
            pltpu.SemaphoreType.DMA,
            pltpu.SemaphoreType.DMA,
            pltpu.SemaphoreType.DMA,
        ],
    )
    def gk(nf_hbm, src_hbm, dst_hbm, xd_out, xs_out,
           idx_d, idx_s, rd0, rd1, rs0, rs1, sd0, sd1, ss0, ss1):
        wid = lax.axis_index("s") * NC + lax.axis_index("c")
        base = wid * per_w
        rd = (rd0, rd1)
        rs = (rs0, rs1)
        sd = (sd0, sd1)
        ss = (ss0, ss1)
        pltpu.sync_copy(dst_hbm.at[pl.ds(e0 + base, per_w)], idx_d)
        pltpu.sync_copy(src_hbm.at[pl.ds(e0 + base, per_w)], idx_s)

        def fire(i, b):
            pltpu.async_copy(nf_hbm.at[idx_d.at[pl.ds(i * chunk, chunk)]],
                             rd[b], sd[b])
            pltpu.async_copy(nf_hbm.at[idx_s.at[pl.ds(i * chunk, chunk)]],
                             rs[b], ss[b])

        def wait(i, b):
            pltpu.make_async_copy(nf_hbm.at[idx_d.at[pl.ds(i * chunk, chunk)]],
                                  rd[b], sd[b]).wait()
            pltpu.make_async_copy(nf_hbm.at[idx_s.at[pl.ds(i * chunk, chunk)]],
                                  rs[b], ss[b]).wait()

        def writeback(i, b):
            off = base + i * chunk
            pltpu.sync_copy(rd[b], xd_out.at[pl.ds(off, chunk)])
            pltpu.sync_copy(rs[b], xs_out.at[pl.ds(off, chunk)])

        fire(0, 0)
        fire(1, 1)

        def body(g, carry):
            i0 = 2 * g
            wait(i0, 0)
            writeback(i0, 0)
            fire(i0 + 2, 0)
            wait(i0 + 1, 1)
            writeback(i0 + 1, 1)
            fire(i0 + 3, 1)
            return carry

        lax.fori_loop(0, g_lim, body, 0)
        i0 = 2 * g_lim
        wait(i0, 0)
        writeback(i0, 0)
        if odd:
            fire(i0 + 2, 0)
        wait(i0 + 1, 1)
        writeback(i0 + 1, 1)
        if odd:
            wait(i0 + 2, 0)
            writeback(i0 + 2, 0)

    return gk(nf, src, dst)


def _sc_scatter(msg, dst3, zeros):
    d = msg.shape[1]
    dt = msg.dtype
    iters, chunk = dst3.shape[1], dst3.shape[2]
    odd = iters % 2 == 1
    g_lim = (iters - 3) // 2 if odd else (iters - 2) // 2
    per_w = iters * chunk
    row_bytes = d * jnp.dtype(dt).itemsize
    tc_tiled = (row_bytes == 512)
    n_acc = 10240 if tc_tiled else N
    rps = n_acc // NS
    mesh = plsc.VectorSubcoreMesh(core_axis_name="c", subcore_axis_name="s")

    @functools.partial(
        pl.kernel,
        out_type=jax.ShapeDtypeStruct((NC, N, d), dt),
        mesh=mesh,
        compiler_params=pltpu.CompilerParams(use_tc_tiling_on_sc=tc_tiled),
        scratch_types=[
            pltpu.VMEM((iters, chunk), jnp.int32),
            pltpu.VMEM((chunk, d), dt),
            pltpu.VMEM((chunk, d), dt),
            pltpu.VMEM_SHARED((n_acc, d), dt),
            pltpu.SemaphoreType.DMA,
            pltpu.SemaphoreType.DMA,
        ],
    )
    def sk(msg_hbm, dst_hbm, z_hbm, out_hbm, idx_v, m0, m1, acc, sm0, sm1):
        cid = lax.axis_index("c")
        sid = lax.axis_index("s")
        wid = sid * NC + cid
        r0 = sid * rps
        pltpu.sync_copy(dst_hbm.at[wid], idx_v)
        pltpu.sync_copy(z_hbm.at[pl.ds(r0, rps)],
                        acc.at[pl.ds(r0, rps)])
        plsc.subcore_barrier()
        base = wid * per_w
        m = (m0, m1)
        sm = (sm0, sm1)

        def fire(i, b):
            pltpu.async_copy(msg_hbm.at[pl.ds(base + i * chunk, chunk)],
                             m[b], sm[b])

        def wait(i, b):
            pltpu.make_async_copy(msg_hbm.at[pl.ds(base + i * chunk, chunk)],
                                  m[b], sm[b]).wait()

        def add(i, b):
            pltpu.sync_copy(m[b], acc.at[idx_v.at[i]], add=True)

        fire(0, 0)
        fire(1, 1)

        def body(g, carry):
            i0 = 2 * g
            wait(i0, 0)
            add(i0, 0)
            fire(i0 + 2, 0)
            wait(i0 + 1, 1)
            add(i0 + 1, 1)
            fire(i0 + 3, 1)
            return carry

        lax.fori_loop(0, g_lim, body, 0)
        i0 = 2 * g_lim
        wait(i0, 0)
        add(i0, 0)
        if odd:
            fire(i0 + 2, 0)
        wait(i0 + 1, 1)
        add(i0 + 1, 1)
        if odd:
            wait(i0 + 2, 0)
            add(i0 + 2, 0)
        plsc.subcore_barrier()
        if n_acc == N:
            pltpu.sync_copy(acc.at[pl.ds(r0, rps)],
                            out_hbm.at[cid, pl.ds(r0, rps)])
        else:
            last = N - (NS - 1) * rps

            @pl.when(sid < NS - 1)
            def _():
                pltpu.sync_copy(acc.at[pl.ds(r0, rps)],
                                out_hbm.at[cid, pl.ds(r0, rps)])

            @pl.when(sid == NS - 1)
            def _():
                pltpu.sync_copy(acc.at[pl.ds((NS - 1) * rps, last)],
                                out_hbm.at[cid, pl.ds((NS - 1) * rps, last)])

    return sk(msg, dst3, zeros)


def _sc_counts(dst, ones, zeros):
    chunk = 1000
    iters = PER_W // chunk
    mesh = plsc.VectorSubcoreMesh(core_axis_name="c", subcore_axis_name="s")

    @functools.partial(
        pl.kernel,
        out_type=jax.ShapeDtypeStruct((NC, N, 8), jnp.float32),
        mesh=mesh,
        compiler_params=pltpu.CompilerParams(use_tc_tiling_on_sc=False),
        scratch_types=[
            pltpu.VMEM((chunk,), jnp.int32),
            pltpu.VMEM((chunk, 8), jnp.float32),
            pltpu.VMEM_SHARED((N, 8), jnp.float32),
        ],
    )
    def ck(dst_hbm, ones_hbm, z_hbm, out_hbm, idx_v, ones_v, acc):
        cid = lax.axis_index("c")
        sid = lax.axis_index("s")
        wid = sid * NC + cid
        r0 = sid * ROWS_PER_SUB
        pltpu.sync_copy(ones_hbm, ones_v)
        pltpu.sync_copy(z_hbm.at[pl.ds(r0, ROWS_PER_SUB)],
                        acc.at[pl.ds(r0, ROWS_PER_SUB)])
        plsc.subcore_barrier()
        base = wid * PER_W

        def body(i, carry):
            off = base + i * chunk
            pltpu.sync_copy(dst_hbm.at[pl.ds(off, chunk)], idx_v)
            pltpu.sync_copy(ones_v, acc.at[idx_v], add=True)
            return carry

        lax.fori_loop(0, iters, body, 0)
        plsc.subcore_barrier()
        pltpu.sync_copy(acc.at[pl.ds(r0, ROWS_PER_SUB)],
                        out_hbm.at[cid, pl.ds(r0, ROWS_PER_SUB)])

    return ck(dst, ones, zeros)


E_A = 96000
E_B = 64000

_SPLIT = ((0, E),)


def _conv_layer(nf, src, dst, p, din):
    w1d, w1s, b1, w2, b2 = _fold_conv(p, din)
    dout = w2.shape[1]
    chunk = {32: 1000, 64: 200, 128: 40}[dout]
    n_acc = 10240 if dout == 128 else N
    parts = []
    for e0, e_cnt in _SPLIT:
        xd, xs = _sc_gather(nf, src, dst, e0, e_cnt)
        msg = _edge_mlp_call(xd, xs, w1d, w1s, b1, w2, b2)
        per_w = e_cnt // NW
        dst3 = lax.dynamic_slice_in_dim(dst, e0, e_cnt).reshape(
            NW, per_w // chunk, chunk)
        parts.append(_sc_scatter(msg, dst3, jnp.zeros((n_acc, dout), msg.dtype)))
    return parts


def kernel(x, matrix, params, batch, edge_index):
    src = edge_index[0]
    dst = edge_index[1]
    batch_col = batch.astype(jnp.float32).reshape(N, 1)

    mg = matrix.astype(jnp.float32).reshape(B, 3, 3)
    zpad = jnp.zeros((B, 1), jnp.float32)
    m0 = jnp.concatenate([zpad, mg[:, 0, :]], axis=1)
    m1 = jnp.concatenate([zpad, mg[:, 1, :]], axis=1)
    m2 = jnp.concatenate([zpad, mg[:, 2, :]], axis=1)

    nf = _embed_call(x, batch_col, m0, m1, m2)

    cnt = _sc_counts(dst, jnp.ones((1000, 8), jnp.float32),
                     jnp.zeros((N, 8), jnp.float32))

    dims = (4, 32, 128, 64, 128)
    for li in range(4):
        p = params[f"conv{li + 1}"]
        (pa,) = _conv_layer(nf, src, dst, p, dims[li])
        ip = params[f"il{li + 1}"]
        k, s = _bn_fold(ip["bn"])
        nf = _node_update_call(
            pa, cnt,
            ip["l1"]["w"].astype(jnp.bfloat16), ip["l1"]["b"][None, :],
            ip["l2"]["w"].astype(jnp.bfloat16), ip["l2"]["b"][None, :],
            k[None, :], s[None, :])

    (pa,) = _conv_layer(nf, src, dst, params["conv5"], dims[4])
    return _pool_fc_call(pa, cnt, batch_col,
                         params["fc1"]["w"], params["fc1"]["b"][None, :],
                         params["fc2"]["w"], params["fc2"]["b"][None, :])


# speedup vs baseline: 1.1505x; 1.0675x over previous
import functools

import jax
import jax.numpy as jnp
from jax import lax
from jax.experimental import pallas as pl
from jax.experimental.pallas import tpu as pltpu
from jax.experimental.pallas import tpu_sc as plsc

N = 10000
E = 160000
B = 128

NC = 2
NS = 16
NW = NC * NS
PER_W = E // NW
ROWS_PER_SUB = N // NS

TILE_E = 4000
TILE_N = 2000


def _lsilu(x, alpha):
    return x * (1.0 / (1.0 + jnp.exp(-x)) + alpha)


def _bn_fold(bn):
    k = bn["g"] / jnp.sqrt(bn["rv"] + 1e-5)
    return k, bn["be"] - bn["rm"] * k


def _fold_conv(p, din):
    k1, s1 = _bn_fold(p["bn1"])
    w1 = (p["l1"]["w"] * k1[None, :]).astype(jnp.bfloat16)
    b1 = p["l1"]["b"] * k1 + s1
    k2, s2 = _bn_fold(p["bn2"])
    w2 = (p["l2"]["w"] * k2[None, :]).astype(jnp.bfloat16)
    b2 = p["l2"]["b"] * k2 + s2
    return w1[:din], w1[din:], b1[None, :], w2, b2[None, :]


def _embed_call(x, batch_col, m0, m1, m2):
    grid = N // TILE_N

    def body(x_ref, b_ref, m0_ref, m1_ref, m2_ref, o_ref):
        xt = x_ref[...]
        bt = b_ref[...]
        iota = lax.broadcasted_iota(jnp.int32, (1, B), 1).astype(jnp.float32)
        oh = jnp.where(bt == iota, 1.0, 0.0)
        col0 = lax.broadcasted_iota(jnp.int32, (1, 4), 1) == 0
        acc = xt[:, 0:1] * jnp.where(col0, 1.0, 0.0)
        for k, m_ref in enumerate((m0_ref, m1_ref, m2_ref)):
            mk = jnp.dot(oh, m_ref[...], preferred_element_type=jnp.float32)
            acc = acc + xt[:, 1 + k:2 + k] * mk
        o_ref[...] = acc

    return pl.pallas_call(
        body,
        grid=(grid,),
        in_specs=[
            pl.BlockSpec((TILE_N, 4), lambda i: (i, 0)),
            pl.BlockSpec((TILE_N, 1), lambda i: (i, 0)),
            pl.BlockSpec((B, 4), lambda i: (0, 0)),
            pl.BlockSpec((B, 4), lambda i: (0, 0)),
            pl.BlockSpec((B, 4), lambda i: (0, 0)),
        ],
        out_specs=pl.BlockSpec((TILE_N, 4), lambda i: (i, 0)),
        out_shape=jax.ShapeDtypeStruct((N, 4), jnp.float32),
    )(x, batch_col, m0, m1, m2)


def _edge_mlp_call(xd, xs, w1d, w1s, b1, w2, b2):
    din = xd.shape[1]
    h_dim = w1d.shape[1]
    dout = w2.shape[1]
    e_cnt = xd.shape[0]
    grid = e_cnt // TILE_E

    def body(xd_ref, xs_ref, w1d_ref, w1s_ref, b1_ref, w2_ref, b2_ref, o_ref):
        h = jnp.dot(xd_ref[...].astype(jnp.bfloat16), w1d_ref[...],
                    preferred_element_type=jnp.float32)
        h = h + jnp.dot(xs_ref[...].astype(jnp.bfloat16), w1s_ref[...],
                        preferred_element_type=jnp.float32)
        h = _lsilu(h + b1_ref[...], 0.05)
        m = jnp.dot(h.astype(jnp.bfloat16), w2_ref[...],
                    preferred_element_type=jnp.float32) + b2_ref[...]
        o_ref[...] = _lsilu(m, 0.05)

    return pl.pallas_call(
        body,
        grid=(grid,),
        in_specs=[
            pl.BlockSpec((TILE_E, din), lambda i: (i, 0)),
            pl.BlockSpec((TILE_E, din), lambda i: (i, 0)),
            pl.BlockSpec((din, h_dim), lambda i: (0, 0)),
            pl.BlockSpec((din, h_dim), lambda i: (0, 0)),
            pl.BlockSpec((1, h_dim), lambda i: (0, 0)),
            pl.BlockSpec((h_dim, dout), lambda i: (0, 0)),
            pl.BlockSpec((1, dout), lambda i: (0, 0)),
        ],
        out_specs=pl.BlockSpec((TILE_E, dout), lambda i: (i, 0)),
        out_shape=jax.ShapeDtypeStruct((e_cnt, dout), jnp.float32),
    )(xd, xs, w1d, w1s, b1, w2, b2)


def _node_update_call(part, cnt, w1, b1, w2, b2, scale, shift):
    d = part.shape[2]
    h_dim = w1.shape[1]
    grid = N // TILE_N

    def body(p_ref, c_ref, w1_ref, b1_ref, w2_ref, b2_ref, sc_ref, sh_ref,
             o_ref):
        s = p_ref[0] + p_ref[1]
        c = c_ref[0, :, 0:1] + c_ref[1, :, 0:1]
        mean = s * (1.0 / jnp.maximum(c, 1.0))
        nf0 = _lsilu(mean, 0.1)
        h = _lsilu(jnp.dot(nf0.astype(jnp.bfloat16), w1_ref[...],
                           preferred_element_type=jnp.float32) + b1_ref[...], 0.05)
        t = _lsilu(jnp.dot(h.astype(jnp.bfloat16), w2_ref[...],
                           preferred_element_type=jnp.float32) + b2_ref[...], 0.05)
        o_ref[...] = t * sc_ref[...] + sh_ref[...] + nf0

    return pl.pallas_call(
        body,
        grid=(grid,),
        in_specs=[
            pl.BlockSpec((2, TILE_N, d), lambda i: (0, i, 0)),
            pl.BlockSpec((2, TILE_N, 8), lambda i: (0, i, 0)),
            pl.BlockSpec((d, h_dim), lambda i: (0, 0)),
            pl.BlockSpec((1, h_dim), lambda i: (0, 0)),
            pl.BlockSpec((h_dim, d), lambda i: (0, 0)),
            pl.BlockSpec((1, d), lambda i: (0, 0)),
            pl.BlockSpec((1, d), lambda i: (0, 0)),
            pl.BlockSpec((1, d), lambda i: (0, 0)),
        ],
        out_specs=pl.BlockSpec((TILE_N, d), lambda i: (i, 0)),
        out_shape=jax.ShapeDtypeStruct((N, d), jnp.float32),
    )(part, cnt, w1, b1, w2, b2, scale, shift)


def _pool_fc_call(part, cnt, batch_col, fw1, fb1, fw2, fb2):
    d = part.shape[2]
    grid = N // TILE_N

    def body(p_ref, c_ref, b_ref, fw1_ref, fb1_ref, fw2_ref, fb2_ref,
             o_ref, acc):
        i = pl.program_id(0)
        s = p_ref[0] + p_ref[1]
        c = c_ref[0, :, 0:1] + c_ref[1, :, 0:1]
        nf = _lsilu(s * (1.0 / jnp.maximum(c, 1.0)), 0.1)
        bt = b_ref[...]
        iota = lax.broadcasted_iota(jnp.int32, (1, B), 1).astype(jnp.float32)
        oh = jnp.where(bt == iota, 1.0, 0.0)
        g = lax.dot_general(oh, nf, (((0,), (0,)), ((), ())),
                            preferred_element_type=jnp.float32)

        @pl.when(i == 0)
        def _():
            acc[...] = jnp.zeros_like(acc)

        acc[...] += g

        @pl.when(i == grid - 1)
        def _():
            h = jnp.dot(acc[...], fw1_ref[...], preferred_element_type=jnp.float32)
            h = _lsilu(h + fb1_ref[...], 0.1)
            o_ref[...] = (jnp.dot(h, fw2_ref[...], preferred_element_type=jnp.float32)
                          + fb2_ref[...] - 100.0)

    return pl.pallas_call(
        body,
        grid=(grid,),
        in_specs=[
            pl.BlockSpec((2, TILE_N, d), lambda i: (0, i, 0)),
            pl.BlockSpec((2, TILE_N, 8), lambda i: (0, i, 0)),
            pl.BlockSpec((TILE_N, 1), lambda i: (i, 0)),
            pl.BlockSpec((d, d), lambda i: (0, 0)),
            pl.BlockSpec((1, d), lambda i: (0, 0)),
            pl.BlockSpec((d, 1), lambda i: (0, 0)),
            pl.BlockSpec((1, 1), lambda i: (0, 0)),
        ],
        out_specs=pl.BlockSpec((B, 1), lambda i: (0, 0)),
        out_shape=jax.ShapeDtypeStruct((B, 1), jnp.float32),
        scratch_shapes=[pltpu.VMEM((B, d), jnp.float32)],
    )(part, cnt, batch_col, fw1, fb1, fw2, fb2)


def _sc_gather(nf, src, dst, e0, e_cnt):
    d = nf.shape[1]
    dt = nf.dtype
    per_w = e_cnt // NW
    row_bytes = d * jnp.dtype(dt).itemsize
    chunk = 1000 if row_bytes <= 64 else 200
    iters = per_w // chunk
    odd = iters % 2 == 1
    g_lim = (iters - 3) // 2 if odd else (iters - 2) // 2
    tc_tiled = False
    mesh = plsc.VectorSubcoreMesh(core_axis_name="c", subcore_axis_name="s")

    @functools.partial(
        pl.kernel,
        out_type=(jax.ShapeDtypeStruct((e_cnt, d), dt),
                  jax.ShapeDtypeStruct((e_cnt, d), dt)),
        mesh=mesh,
        compiler_params=pltpu.CompilerParams(use_tc_tiling_on_sc=tc_tiled),
        scratch_types=[
            pltpu.VMEM((per_w,), jnp.int32),
            pltpu.VMEM((per_w,), jnp.int32),
            pltpu.VMEM((chunk, d), dt),
            pltpu.VMEM((chunk, d), dt),
            pltpu.VMEM((chunk, d), dt),
            pltpu.VMEM((chunk, d), dt),
            pltpu.SemaphoreType.DMA,
            pltpu.SemaphoreType.DMA,
            pltpu.SemaphoreType.DMA,
            pltpu.SemaphoreType.DMA,
        ],
    )
    def gk(nf_hbm, src_hbm, dst_hbm, xd_out, xs_out,
           idx_d, idx_s, rd0, rd1, rs0, rs1, sd0, sd1, ss0, ss1):
        wid = lax.axis_index("s") * NC + lax.axis_index("c")
        base = wid * per_w
        rd = (rd0, rd1)
        rs = (rs0, rs1)
        sd = (sd0, sd1)
        ss = (ss0, ss1)
        pltpu.sync_copy(dst_hbm.at[pl.ds(e0 + base, per_w)], idx_d)
        pltpu.sync_copy(src_hbm.at[pl.ds(e0 + base, per_w)], idx_s)

        def fire(i, b):
            pltpu.async_copy(nf_hbm.at[idx_d.at[pl.ds(i * chunk, chunk)]],
                             rd[b], sd[b])
            pltpu.async_copy(nf_hbm.at[idx_s.at[pl.ds(i * chunk, chunk)]],
                             rs[b], ss[b])

        def wait(i, b):
            pltpu.make_async_copy(nf_hbm.at[idx_d.at[pl.ds(i * chunk, chunk)]],
                                  rd[b], sd[b]).wait()
            pltpu.make_async_copy(nf_hbm.at[idx_s.at[pl.ds(i * chunk, chunk)]],
                                  rs[b], ss[b]).wait()

        def writeback(i, b):
            off = base + i * chunk
            pltpu.sync_copy(rd[b], xd_out.at[pl.ds(off, chunk)])
            pltpu.sync_copy(rs[b], xs_out.at[pl.ds(off, chunk)])

        fire(0, 0)
        fire(1, 1)

        def body(g, carry):
            i0 = 2 * g
            wait(i0, 0)
            writeback(i0, 0)
            fire(i0 + 2, 0)
            wait(i0 + 1, 1)
            writeback(i0 + 1, 1)
            fire(i0 + 3, 1)
            return carry

        lax.fori_loop(0, g_lim, body, 0)
        i0 = 2 * g_lim
        wait(i0, 0)
        writeback(i0, 0)
        if odd:
            fire(i0 + 2, 0)
        wait(i0 + 1, 1)
        writeback(i0 + 1, 1)
        if odd:
            wait(i0 + 2, 0)
            writeback(i0 + 2, 0)

    return gk(nf, src, dst)


def _sc_scatter(msg, dst3, zeros):
    d = msg.shape[1]
    dt = msg.dtype
    iters, chunk = dst3.shape[1], dst3.shape[2]
    odd = iters % 2 == 1
    g_lim = (iters - 3) // 2 if odd else (iters - 2) // 2
    per_w = iters * chunk
    row_bytes = d * jnp.dtype(dt).itemsize
    tc_tiled = False
    n_acc = 10240 if tc_tiled else N
    rps = n_acc // NS
    mesh = plsc.VectorSubcoreMesh(core_axis_name="c", subcore_axis_name="s")

    @functools.partial(
        pl.kernel,
        out_type=jax.ShapeDtypeStruct((NC, N, d), dt),
        mesh=mesh,
        compiler_params=pltpu.CompilerParams(use_tc_tiling_on_sc=tc_tiled),
        scratch_types=[
            pltpu.VMEM((iters, chunk), jnp.int32),
            pltpu.VMEM((chunk, d), dt),
            pltpu.VMEM((chunk, d), dt),
            pltpu.VMEM_SHARED((n_acc, d), dt),
            pltpu.SemaphoreType.DMA,
            pltpu.SemaphoreType.DMA,
        ],
    )
    def sk(msg_hbm, dst_hbm, z_hbm, out_hbm, idx_v, m0, m1, acc, sm0, sm1):
        cid = lax.axis_index("c")
        sid = lax.axis_index("s")
        wid = sid * NC + cid
        r0 = sid * rps
        pltpu.sync_copy(dst_hbm.at[wid], idx_v)
        pltpu.sync_copy(z_hbm.at[pl.ds(r0, rps)],
                        acc.at[pl.ds(r0, rps)])
        plsc.subcore_barrier()
        base = wid * per_w
        m = (m0, m1)
        sm = (sm0, sm1)

        def fire(i, b):
            pltpu.async_copy(msg_hbm.at[pl.ds(base + i * chunk, chunk)],
                             m[b], sm[b])

        def wait(i, b):
            pltpu.make_async_copy(msg_hbm.at[pl.ds(base + i * chunk, chunk)],
                                  m[b], sm[b]).wait()

        def add(i, b):
            pltpu.sync_copy(m[b], acc.at[idx_v.at[i]], add=True)

        fire(0, 0)
        fire(1, 1)

        def body(g, carry):
            i0 = 2 * g
            wait(i0, 0)
            add(i0, 0)
            fire(i0 + 2, 0)
            wait(i0 + 1, 1)
            add(i0 + 1, 1)
            fire(i0 + 3, 1)
            return carry

        lax.fori_loop(0, g_lim, body, 0)
        i0 = 2 * g_lim
        wait(i0, 0)
        add(i0, 0)
        if odd:
            fire(i0 + 2, 0)
        wait(i0 + 1, 1)
        add(i0 + 1, 1)
        if odd:
            wait(i0 + 2, 0)
            add(i0 + 2, 0)
        plsc.subcore_barrier()
        if n_acc == N:
            pltpu.sync_copy(acc.at[pl.ds(r0, rps)],
                            out_hbm.at[cid, pl.ds(r0, rps)])
        else:
            last = N - (NS - 1) * rps

            @pl.when(sid < NS - 1)
            def _():
                pltpu.sync_copy(acc.at[pl.ds(r0, rps)],
                                out_hbm.at[cid, pl.ds(r0, rps)])

            @pl.when(sid == NS - 1)
            def _():
                pltpu.sync_copy(acc.at[pl.ds((NS - 1) * rps, last)],
                                out_hbm.at[cid, pl.ds((NS - 1) * rps, last)])

    return sk(msg, dst3, zeros)


def _sc_counts(dst, ones, zeros):
    chunk = 1000
    iters = PER_W // chunk
    mesh = plsc.VectorSubcoreMesh(core_axis_name="c", subcore_axis_name="s")

    @functools.partial(
        pl.kernel,
        out_type=jax.ShapeDtypeStruct((NC, N, 8), jnp.float32),
        mesh=mesh,
        compiler_params=pltpu.CompilerParams(use_tc_tiling_on_sc=False),
        scratch_types=[
            pltpu.VMEM((chunk,), jnp.int32),
            pltpu.VMEM((chunk, 8), jnp.float32),
            pltpu.VMEM_SHARED((N, 8), jnp.float32),
        ],
    )
    def ck(dst_hbm, ones_hbm, z_hbm, out_hbm, idx_v, ones_v, acc):
        cid = lax.axis_index("c")
        sid = lax.axis_index("s")
        wid = sid * NC + cid
        r0 = sid * ROWS_PER_SUB
        pltpu.sync_copy(ones_hbm, ones_v)
        pltpu.sync_copy(z_hbm.at[pl.ds(r0, ROWS_PER_SUB)],
                        acc.at[pl.ds(r0, ROWS_PER_SUB)])
        plsc.subcore_barrier()
        base = wid * PER_W

        def body(i, carry):
            off = base + i * chunk
            pltpu.sync_copy(dst_hbm.at[pl.ds(off, chunk)], idx_v)
            pltpu.sync_copy(ones_v, acc.at[idx_v], add=True)
            return carry

        lax.fori_loop(0, iters, body, 0)
        plsc.subcore_barrier()
        pltpu.sync_copy(acc.at[pl.ds(r0, ROWS_PER_SUB)],
                        out_hbm.at[cid, pl.ds(r0, ROWS_PER_SUB)])

    return ck(dst, ones, zeros)


E_A = 96000
E_B = 64000

_SPLIT = ((0, E),)


def _conv_layer(nf, src, dst, p, din):
    w1d, w1s, b1, w2, b2 = _fold_conv(p, din)
    dout = w2.shape[1]
    chunk = {32: 1000, 64: 200, 128: 40}[dout]
    n_acc = 10240 if dout == 128 else N
    parts = []
    for e0, e_cnt in _SPLIT:
        xd, xs = _sc_gather(nf, src, dst, e0, e_cnt)
        msg = _edge_mlp_call(xd, xs, w1d, w1s, b1, w2, b2)
        per_w = e_cnt // NW
        dst3 = lax.dynamic_slice_in_dim(dst, e0, e_cnt).reshape(
            NW, per_w // chunk, chunk)
        parts.append(_sc_scatter(msg, dst3, jnp.zeros((n_acc, dout), msg.dtype)))
    return parts


def kernel(x, matrix, params, batch, edge_index):
    src = edge_index[0]
    dst = edge_index[1]
    batch_col = batch.astype(jnp.float32).reshape(N, 1)

    mg = matrix.astype(jnp.float32).reshape(B, 3, 3)
    zpad = jnp.zeros((B, 1), jnp.float32)
    m0 = jnp.concatenate([zpad, mg[:, 0, :]], axis=1)
    m1 = jnp.concatenate([zpad, mg[:, 1, :]], axis=1)
    m2 = jnp.concatenate([zpad, mg[:, 2, :]], axis=1)

    nf = _embed_call(x, batch_col, m0, m1, m2)

    cnt = _sc_counts(dst, jnp.ones((1000, 8), jnp.float32),
                     jnp.zeros((N, 8), jnp.float32))

    dims = (4, 32, 128, 64, 128)
    for li in range(4):
        p = params[f"conv{li + 1}"]
        (pa,) = _conv_layer(nf, src, dst, p, dims[li])
        ip = params[f"il{li + 1}"]
        k, s = _bn_fold(ip["bn"])
        nf = _node_update_call(
            pa, cnt,
            ip["l1"]["w"].astype(jnp.bfloat16), ip["l1"]["b"][None, :],
            ip["l2"]["w"].astype(jnp.bfloat16), ip["l2"]["b"][None, :],
            k[None, :], s[None, :])

    (pa,) = _conv_layer(nf, src, dst, params["conv5"], dims[4])
    return _pool_fc_call(pa, cnt, batch_col,
                         params["fc1"]["w"], params["fc1"]["b"][None, :],
                         params["fc2"]["w"], params["fc2"]["b"][None, :])


# speedup vs baseline: 1.1781x; 1.0240x over previous
import functools

import jax
import jax.numpy as jnp
from jax import lax
from jax.experimental import pallas as pl
from jax.experimental.pallas import tpu as pltpu
from jax.experimental.pallas import tpu_sc as plsc

N = 10000
E = 160000
B = 128

NC = 2
NS = 16
NW = NC * NS
PER_W = E // NW
ROWS_PER_SUB = N // NS

TILE_E = 8000
TILE_N = 2000


def _lsilu(x, alpha):
    return x * (1.0 / (1.0 + jnp.exp(-x)) + alpha)


def _bn_fold(bn):
    k = bn["g"] / jnp.sqrt(bn["rv"] + 1e-5)
    return k, bn["be"] - bn["rm"] * k


def _fold_conv(p, din):
    k1, s1 = _bn_fold(p["bn1"])
    w1 = (p["l1"]["w"] * k1[None, :]).astype(jnp.bfloat16)
    b1 = p["l1"]["b"] * k1 + s1
    k2, s2 = _bn_fold(p["bn2"])
    w2 = (p["l2"]["w"] * k2[None, :]).astype(jnp.bfloat16)
    b2 = p["l2"]["b"] * k2 + s2
    return w1[:din], w1[din:], b1[None, :], w2, b2[None, :]


def _embed_call(x, batch_col, m0, m1, m2):
    grid = N // TILE_N

    def body(x_ref, b_ref, m0_ref, m1_ref, m2_ref, o_ref):
        xt = x_ref[...]
        bt = b_ref[...]
        iota = lax.broadcasted_iota(jnp.int32, (1, B), 1).astype(jnp.float32)
        oh = jnp.where(bt == iota, 1.0, 0.0)
        col0 = lax.broadcasted_iota(jnp.int32, (1, 4), 1) == 0
        acc = xt[:, 0:1] * jnp.where(col0, 1.0, 0.0)
        for k, m_ref in enumerate((m0_ref, m1_ref, m2_ref)):
            mk = jnp.dot(oh, m_ref[...], preferred_element_type=jnp.float32)
            acc = acc + xt[:, 1 + k:2 + k] * mk
        o_ref[...] = acc

    return pl.pallas_call(
        body,
        grid=(grid,),
        in_specs=[
            pl.BlockSpec((TILE_N, 4), lambda i: (i, 0)),
            pl.BlockSpec((TILE_N, 1), lambda i: (i, 0)),
            pl.BlockSpec((B, 4), lambda i: (0, 0)),
            pl.BlockSpec((B, 4), lambda i: (0, 0)),
            pl.BlockSpec((B, 4), lambda i: (0, 0)),
        ],
        out_specs=pl.BlockSpec((TILE_N, 4), lambda i: (i, 0)),
        out_shape=jax.ShapeDtypeStruct((N, 4), jnp.float32),
    )(x, batch_col, m0, m1, m2)


def _edge_mlp_call(xd, xs, w1d, w1s, b1, w2, b2):
    din = xd.shape[1]
    h_dim = w1d.shape[1]
    dout = w2.shape[1]
    e_cnt = xd.shape[0]
    grid = e_cnt // TILE_E

    def body(xd_ref, xs_ref, w1d_ref, w1s_ref, b1_ref, w2_ref, b2_ref, o_ref):
        h = jnp.dot(xd_ref[...].astype(jnp.bfloat16), w1d_ref[...],
                    preferred_element_type=jnp.float32)
        h = h + jnp.dot(xs_ref[...].astype(jnp.bfloat16), w1s_ref[...],
                        preferred_element_type=jnp.float32)
        h = _lsilu(h + b1_ref[...], 0.05)
        m = jnp.dot(h.astype(jnp.bfloat16), w2_ref[...],
                    preferred_element_type=jnp.float32) + b2_ref[...]
        o_ref[...] = _lsilu(m, 0.05)

    return pl.pallas_call(
        body,
        grid=(grid,),
        in_specs=[
            pl.BlockSpec((TILE_E, din), lambda i: (i, 0)),
            pl.BlockSpec((TILE_E, din), lambda i: (i, 0)),
            pl.BlockSpec((din, h_dim), lambda i: (0, 0)),
            pl.BlockSpec((din, h_dim), lambda i: (0, 0)),
            pl.BlockSpec((1, h_dim), lambda i: (0, 0)),
            pl.BlockSpec((h_dim, dout), lambda i: (0, 0)),
            pl.BlockSpec((1, dout), lambda i: (0, 0)),
        ],
        out_specs=pl.BlockSpec((TILE_E, dout), lambda i: (i, 0)),
        out_shape=jax.ShapeDtypeStruct((e_cnt, dout), jnp.float32),
    )(xd, xs, w1d, w1s, b1, w2, b2)


def _node_update_call(part, cnt, w1, b1, w2, b2, scale, shift):
    d = part.shape[2]
    h_dim = w1.shape[1]
    grid = N // TILE_N

    def body(p_ref, c_ref, w1_ref, b1_ref, w2_ref, b2_ref, sc_ref, sh_ref,
             o_ref):
        s = p_ref[0] + p_ref[1]
        c = c_ref[0, :, 0:1] + c_ref[1, :, 0:1]
        mean = s * (1.0 / jnp.maximum(c, 1.0))
        nf0 = _lsilu(mean, 0.1)
        h = _lsilu(jnp.dot(nf0.astype(jnp.bfloat16), w1_ref[...],
                           preferred_element_type=jnp.float32) + b1_ref[...], 0.05)
        t = _lsilu(jnp.dot(h.astype(jnp.bfloat16), w2_ref[...],
                           preferred_element_type=jnp.float32) + b2_ref[...], 0.05)
        o_ref[...] = t * sc_ref[...] + sh_ref[...] + nf0

    return pl.pallas_call(
        body,
        grid=(grid,),
        in_specs=[
            pl.BlockSpec((2, TILE_N, d), lambda i: (0, i, 0)),
            pl.BlockSpec((2, TILE_N, 8), lambda i: (0, i, 0)),
            pl.BlockSpec((d, h_dim), lambda i: (0, 0)),
            pl.BlockSpec((1, h_dim), lambda i: (0, 0)),
            pl.BlockSpec((h_dim, d), lambda i: (0, 0)),
            pl.BlockSpec((1, d), lambda i: (0, 0)),
            pl.BlockSpec((1, d), lambda i: (0, 0)),
            pl.BlockSpec((1, d), lambda i: (0, 0)),
        ],
        out_specs=pl.BlockSpec((TILE_N, d), lambda i: (i, 0)),
        out_shape=jax.ShapeDtypeStruct((N, d), jnp.float32),
    )(part, cnt, w1, b1, w2, b2, scale, shift)


def _pool_fc_call(part, cnt, batch_col, fw1, fb1, fw2, fb2):
    d = part.shape[2]
    grid = N // TILE_N

    def body(p_ref, c_ref, b_ref, fw1_ref, fb1_ref, fw2_ref, fb2_ref,
             o_ref, acc):
        i = pl.program_id(0)
        s = p_ref[0] + p_ref[1]
        c = c_ref[0, :, 0:1] + c_ref[1, :, 0:1]
        nf = _lsilu(s * (1.0 / jnp.maximum(c, 1.0)), 0.1)
        bt = b_ref[...]
        iota = lax.broadcasted_iota(jnp.int32, (1, B), 1).astype(jnp.float32)
        oh = jnp.where(bt == iota, 1.0, 0.0)
        g = lax.dot_general(oh, nf, (((0,), (0,)), ((), ())),
                            preferred_element_type=jnp.float32)

        @pl.when(i == 0)
        def _():
            acc[...] = jnp.zeros_like(acc)

        acc[...] += g

        @pl.when(i == grid - 1)
        def _():
            h = jnp.dot(acc[...], fw1_ref[...], preferred_element_type=jnp.float32)
            h = _lsilu(h + fb1_ref[...], 0.1)
            o_ref[...] = (jnp.dot(h, fw2_ref[...], preferred_element_type=jnp.float32)
                          + fb2_ref[...] - 100.0)

    return pl.pallas_call(
        body,
        grid=(grid,),
        in_specs=[
            pl.BlockSpec((2, TILE_N, d), lambda i: (0, i, 0)),
            pl.BlockSpec((2, TILE_N, 8), lambda i: (0, i, 0)),
            pl.BlockSpec((TILE_N, 1), lambda i: (i, 0)),
            pl.BlockSpec((d, d), lambda i: (0, 0)),
            pl.BlockSpec((1, d), lambda i: (0, 0)),
            pl.BlockSpec((d, 1), lambda i: (0, 0)),
            pl.BlockSpec((1, 1), lambda i: (0, 0)),
        ],
        out_specs=pl.BlockSpec((B, 1), lambda i: (0, 0)),
        out_shape=jax.ShapeDtypeStruct((B, 1), jnp.float32),
        scratch_shapes=[pltpu.VMEM((B, d), jnp.float32)],
    )(part, cnt, batch_col, fw1, fb1, fw2, fb2)


def _sc_gather(nf, src, dst, e0, e_cnt):
    d = nf.shape[1]
    dt = nf.dtype
    per_w = e_cnt // NW
    row_bytes = d * jnp.dtype(dt).itemsize
    chunk = 1000 if row_bytes <= 64 else 200
    iters = per_w // chunk
    odd = iters % 2 == 1
    g_lim = (iters - 3) // 2 if odd else (iters - 2) // 2
    tc_tiled = False
    mesh = plsc.VectorSubcoreMesh(core_axis_name="c", subcore_axis_name="s")

    @functools.partial(
        pl.kernel,
        out_type=(jax.ShapeDtypeStruct((e_cnt, d), dt),
                  jax.ShapeDtypeStruct((e_cnt, d), dt)),
        mesh=mesh,
        compiler_params=pltpu.CompilerParams(use_tc_tiling_on_sc=tc_tiled),
        scratch_types=[
            pltpu.VMEM((per_w,), jnp.int32),
            pltpu.VMEM((per_w,), jnp.int32),
            pltpu.VMEM((chunk, d), dt),
            pltpu.VMEM((chunk, d), dt),
            pltpu.VMEM((chunk, d), dt),
            pltpu.VMEM((chunk, d), dt),
            pltpu.SemaphoreType.DMA,
            pltpu.SemaphoreType.DMA,
            pltpu.SemaphoreType.DMA,
            pltpu.SemaphoreType.DMA,
        ],
    )
    def gk(nf_hbm, src_hbm, dst_hbm, xd_out, xs_out,
           idx_d, idx_s, rd0, rd1, rs0, rs1, sd0, sd1, ss0, ss1):
        wid = lax.axis_index("s") * NC + lax.axis_index("c")
        base = wid * per_w
        rd = (rd0, rd1)
        rs = (rs0, rs1)
        sd = (sd0, sd1)
        ss = (ss0, ss1)
        pltpu.sync_copy(dst_hbm.at[pl.ds(e0 + base, per_w)], idx_d)
        pltpu.sync_copy(src_hbm.at[pl.ds(e0 + base, per_w)], idx_s)

        def fire(i, b):
            pltpu.async_copy(nf_hbm.at[idx_d.at[pl.ds(i * chunk, chunk)]],
                             rd[b], sd[b])
            pltpu.async_copy(nf_hbm.at[idx_s.at[pl.ds(i * chunk, chunk)]],
                             rs[b], ss[b])

        def wait(i, b):
            pltpu.make_async_copy(nf_hbm.at[idx_d.at[pl.ds(i * chunk, chunk)]],
                                  rd[b], sd[b]).wait()
            pltpu.make_async_copy(nf_hbm.at[idx_s.at[pl.ds(i * chunk, chunk)]],
                                  rs[b], ss[b]).wait()

        def writeback(i, b):
            off = base + i * chunk
            pltpu.sync_copy(rd[b], xd_out.at[pl.ds(off, chunk)])
            pltpu.sync_copy(rs[b], xs_out.at[pl.ds(off, chunk)])

        fire(0, 0)
        fire(1, 1)

        def body(g, carry):
            i0 = 2 * g
            wait(i0, 0)
            writeback(i0, 0)
            fire(i0 + 2, 0)
            wait(i0 + 1, 1)
            writeback(i0 + 1, 1)
            fire(i0 + 3, 1)
            return carry

        lax.fori_loop(0, g_lim, body, 0)
        i0 = 2 * g_lim
        wait(i0, 0)
        writeback(i0, 0)
        if odd:
            fire(i0 + 2, 0)
        wait(i0 + 1, 1)
        writeback(i0 + 1, 1)
        if odd:
            wait(i0 + 2, 0)
            writeback(i0 + 2, 0)

    return gk(nf, src, dst)


def _sc_scatter(msg, dst3, zeros):
    d = msg.shape[1]
    dt = msg.dtype
    iters, chunk = dst3.shape[1], dst3.shape[2]
    odd = iters % 2 == 1
    g_lim = (iters - 3) // 2 if odd else (iters - 2) // 2
    per_w = iters * chunk
    row_bytes = d * jnp.dtype(dt).itemsize
    tc_tiled = False
    n_acc = 10240 if tc_tiled else N
    rps = n_acc // NS
    mesh = plsc.VectorSubcoreMesh(core_axis_name="c", subcore_axis_name="s")

    @functools.partial(
        pl.kernel,
        out_type=jax.ShapeDtypeStruct((NC, N, d), dt),
        mesh=mesh,
        compiler_params=pltpu.CompilerParams(use_tc_tiling_on_sc=tc_tiled),
        scratch_types=[
            pltpu.VMEM((iters, chunk), jnp.int32),
            pltpu.VMEM((chunk, d), dt),
            pltpu.VMEM((chunk, d), dt),
            pltpu.VMEM_SHARED((n_acc, d), dt),
            pltpu.SemaphoreType.DMA,
            pltpu.SemaphoreType.DMA,
        ],
    )
    def sk(msg_hbm, dst_hbm, z_hbm, out_hbm, idx_v, m0, m1, acc, sm0, sm1):
        cid = lax.axis_index("c")
        sid = lax.axis_index("s")
        wid = sid * NC + cid
        r0 = sid * rps
        pltpu.sync_copy(dst_hbm.at[wid], idx_v)
        pltpu.sync_copy(z_hbm.at[pl.ds(r0, rps)],
                        acc.at[pl.ds(r0, rps)])
        plsc.subcore_barrier()
        base = wid * per_w
        m = (m0, m1)
        sm = (sm0, sm1)

        def fire(i, b):
            pltpu.async_copy(msg_hbm.at[pl.ds(base + i * chunk, chunk)],
                             m[b], sm[b])

        def wait(i, b):
            pltpu.make_async_copy(msg_hbm.at[pl.ds(base + i * chunk, chunk)],
                                  m[b], sm[b]).wait()

        def add(i, b):
            pltpu.sync_copy(m[b], acc.at[idx_v.at[i]], add=True)

        fire(0, 0)
        fire(1, 1)

        def body(g, carry):
            i0 = 2 * g
            wait(i0, 0)
            add(i0, 0)
            fire(i0 + 2, 0)
            wait(i0 + 1, 1)
            add(i0 + 1, 1)
            fire(i0 + 3, 1)
            return carry

        lax.fori_loop(0, g_lim, body, 0)
        i0 = 2 * g_lim
        wait(i0, 0)
        add(i0, 0)
        if odd:
            fire(i0 + 2, 0)
        wait(i0 + 1, 1)
        add(i0 + 1, 1)
        if odd:
            wait(i0 + 2, 0)
            add(i0 + 2, 0)
        plsc.subcore_barrier()
        if n_acc == N:
            pltpu.sync_copy(acc.at[pl.ds(r0, rps)],
                            out_hbm.at[cid, pl.ds(r0, rps)])
        else:
            last = N - (NS - 1) * rps

            @pl.when(sid < NS - 1)
            def _():
                pltpu.sync_copy(acc.at[pl.ds(r0, rps)],
                                out_hbm.at[cid, pl.ds(r0, rps)])

            @pl.when(sid == NS - 1)
            def _():
                pltpu.sync_copy(acc.at[pl.ds((NS - 1) * rps, last)],
                                out_hbm.at[cid, pl.ds((NS - 1) * rps, last)])

    return sk(msg, dst3, zeros)


def _sc_counts(dst, ones, zeros):
    chunk = 1000
    iters = PER_W // chunk
    mesh = plsc.VectorSubcoreMesh(core_axis_name="c", subcore_axis_name="s")

    @functools.partial(
        pl.kernel,
        out_type=jax.ShapeDtypeStruct((NC, N, 8), jnp.float32),
        mesh=mesh,
        compiler_params=pltpu.CompilerParams(use_tc_tiling_on_sc=False),
        scratch_types=[
            pltpu.VMEM((chunk,), jnp.int32),
            pltpu.VMEM((chunk, 8), jnp.float32),
            pltpu.VMEM_SHARED((N, 8), jnp.float32),
        ],
    )
    def ck(dst_hbm, ones_hbm, z_hbm, out_hbm, idx_v, ones_v, acc):
        cid = lax.axis_index("c")
        sid = lax.axis_index("s")
        wid = sid * NC + cid
        r0 = sid * ROWS_PER_SUB
        pltpu.sync_copy(ones_hbm, ones_v)
        pltpu.sync_copy(z_hbm.at[pl.ds(r0, ROWS_PER_SUB)],
                        acc.at[pl.ds(r0, ROWS_PER_SUB)])
        plsc.subcore_barrier()
        base = wid * PER_W

        def body(i, carry):
            off = base + i * chunk
            pltpu.sync_copy(dst_hbm.at[pl.ds(off, chunk)], idx_v)
            pltpu.sync_copy(ones_v, acc.at[idx_v], add=True)
            return carry

        lax.fori_loop(0, iters, body, 0)
        plsc.subcore_barrier()
        pltpu.sync_copy(acc.at[pl.ds(r0, ROWS_PER_SUB)],
                        out_hbm.at[cid, pl.ds(r0, ROWS_PER_SUB)])

    return ck(dst, ones, zeros)


E_A = 96000
E_B = 64000

_SPLIT = ((0, E),)


def _conv_layer(nf, src, dst, p, din):
    w1d, w1s, b1, w2, b2 = _fold_conv(p, din)
    dout = w2.shape[1]
    chunk = {32: 1000, 64: 200, 128: 40}[dout]
    n_acc = 10240 if dout == 128 else N
    parts = []
    for e0, e_cnt in _SPLIT:
        xd, xs = _sc_gather(nf, src, dst, e0, e_cnt)
        msg = _edge_mlp_call(xd, xs, w1d, w1s, b1, w2, b2)
        per_w = e_cnt // NW
        dst3 = lax.dynamic_slice_in_dim(dst, e0, e_cnt).reshape(
            NW, per_w // chunk, chunk)
        parts.append(_sc_scatter(msg, dst3, jnp.zeros((n_acc, dout), msg.dtype)))
    return parts


def kernel(x, matrix, params, batch, edge_index):
    src = edge_index[0]
    dst = edge_index[1]
    batch_col = batch.astype(jnp.float32).reshape(N, 1)

    mg = matrix.astype(jnp.float32).reshape(B, 3, 3)
    zpad = jnp.zeros((B, 1), jnp.float32)
    m0 = jnp.concatenate([zpad, mg[:, 0, :]], axis=1)
    m1 = jnp.concatenate([zpad, mg[:, 1, :]], axis=1)
    m2 = jnp.concatenate([zpad, mg[:, 2, :]], axis=1)

    nf = _embed_call(x, batch_col, m0, m1, m2)

    cnt = _sc_counts(dst, jnp.ones((1000, 8), jnp.float32),
                     jnp.zeros((N, 8), jnp.float32))

    dims = (4, 32, 128, 64, 128)
    for li in range(4):
        p = params[f"conv{li + 1}"]
        (pa,) = _conv_layer(nf, src, dst, p, dims[li])
        ip = params[f"il{li + 1}"]
        k, s = _bn_fold(ip["bn"])
        nf = _node_update_call(
            pa, cnt,
            ip["l1"]["w"].astype(jnp.bfloat16), ip["l1"]["b"][None, :],
            ip["l2"]["w"].astype(jnp.bfloat16), ip["l2"]["b"][None, :],
            k[None, :], s[None, :])

    (pa,) = _conv_layer(nf, src, dst, params["conv5"], dims[4])
    return _pool_fc_call(pa, cnt, batch_col,
                         params["fc1"]["w"], params["fc1"]["b"][None, :],
                         params["fc2"]["w"], params["fc2"]["b"][None, :])
